# Initial kernel scaffold; baseline (speedup 1.0000x reference)
import jax
import jax.numpy as jnp
from jax.experimental import pallas as pl


def kernel(x, adj, W1, b1, W2, b2):
    raise NotImplementedError("write your pallas kernel here")


# speedup vs baseline: 5.6543x; 5.6543x over previous
import functools

import jax
import jax.numpy as jnp
from jax import lax
from jax.experimental import pallas as pl
from jax.experimental.pallas import tpu as pltpu
from jax.experimental.pallas import tpu_sc as plsc

NC = 2
NS = 16
NW = NC * NS
CHUNK = 128


def _mm_bias_block(x_ref, w_ref, b_ref, o_ref):
    o_ref[...] = (
        jnp.dot(x_ref[...], w_ref[...], preferred_element_type=jnp.float32)
        + b_ref[...]
    )


@functools.lru_cache(maxsize=None)
def _make_mm_bias(n, k, h, bm):
    return pl.pallas_call(
        _mm_bias_block,
        grid=(n // bm,),
        in_specs=[
            pl.BlockSpec((bm, k), lambda i: (i, 0)),
            pl.BlockSpec((k, h), lambda i: (0, 0)),
            pl.BlockSpec((1, h), lambda i: (0, 0)),
        ],
        out_specs=pl.BlockSpec((bm, h), lambda i: (i, 0)),
        out_shape=jax.ShapeDtypeStruct((n, h), jnp.float32),
    )


def _fused_relu_mm_block(a0_ref, a1_ref, w_ref, b_ref, o_ref):
    h = jnp.maximum(a0_ref[...] + a1_ref[...], 0.0)
    o_ref[...] = (
        jnp.dot(h, w_ref[...], preferred_element_type=jnp.float32) + b_ref[...]
    )


@functools.lru_cache(maxsize=None)
def _make_fused_relu_mm(n, k, h, bm):
    return pl.pallas_call(
        _fused_relu_mm_block,
        grid=(n // bm,),
        in_specs=[
            pl.BlockSpec((bm, k), lambda i: (i, 0)),
            pl.BlockSpec((bm, k), lambda i: (i, 0)),
            pl.BlockSpec((k, h), lambda i: (0, 0)),
            pl.BlockSpec((1, h), lambda i: (0, 0)),
        ],
        out_specs=pl.BlockSpec((bm, h), lambda i: (i, 0)),
        out_shape=jax.ShapeDtypeStruct((n, h), jnp.float32),
    )


def _add_block(a_ref, b_ref, o_ref):
    o_ref[...] = a_ref[...] + b_ref[...]


@functools.lru_cache(maxsize=None)
def _make_add(n, h, bm):
    return pl.pallas_call(
        _add_block,
        grid=(n // bm,),
        in_specs=[
            pl.BlockSpec((bm, h), lambda i: (i, 0)),
            pl.BlockSpec((bm, h), lambda i: (i, 0)),
        ],
        out_specs=pl.BlockSpec((bm, h), lambda i: (i, 0)),
        out_shape=jax.ShapeDtypeStruct((n, h), jnp.float32),
    )


@functools.lru_cache(maxsize=None)
def _make_sc_agg(n_nodes, n_edges, d):
    e_per_w = n_edges // NW
    assert e_per_w * NW == n_edges
    n_full = e_per_w // CHUNK
    tail = e_per_w - n_full * CHUNK
    assert tail % 8 == 0 and e_per_w % 8 == 0
    slab = (n_nodes // NS) // 8 * 8
    rem = n_nodes - NS * slab
    assert rem % 8 == 0

    mesh = plsc.VectorSubcoreMesh(core_axis_name="c", subcore_axis_name="s")

    scratch = [
        pltpu.VMEM((CHUNK,), jnp.int32),
        pltpu.VMEM((CHUNK,), jnp.int32),
        pltpu.VMEM((CHUNK, d), jnp.float32),
        pltpu.VMEM_SHARED((n_nodes, d), jnp.float32),
        pltpu.SemaphoreType.DMA,
    ]
    if tail:
        scratch += [
            pltpu.VMEM((tail,), jnp.int32),
            pltpu.VMEM((tail,), jnp.int32),
            pltpu.VMEM((tail, d), jnp.float32),
        ]

    @functools.partial(
        pl.kernel,
        out_type=jax.ShapeDtypeStruct((NC * n_nodes, d), jnp.float32),
        mesh=mesh,
        scratch_types=scratch,
        compiler_params=pltpu.CompilerParams(use_tc_tiling_on_sc=False),
    )
    def agg(h_hbm, src_hbm, dst_hbm, zeros_hbm, out_hbm,
            src_v, dst_v, rows_v, acc_sh, sem, *tail_bufs):
        c = lax.axis_index("c")
        s = lax.axis_index("s")
        wid = s * NC + c
        row0 = s * slab
        pltpu.sync_copy(zeros_hbm.at[pl.ds(row0, slab)],
                        acc_sh.at[pl.ds(row0, slab)])
        if rem:
            @pl.when(s == NS - 1)
            def _():
                pltpu.sync_copy(zeros_hbm.at[pl.ds(NS * slab, rem)],
                                acc_sh.at[pl.ds(NS * slab, rem)])
        plsc.subcore_barrier()

        e0 = wid * e_per_w

        def step(j, carry):
            base = e0 + j * CHUNK
            pltpu.sync_copy(src_hbm.at[pl.ds(base, CHUNK)], src_v)
            pltpu.async_copy(h_hbm.at[src_v], rows_v, sem).wait()
            pltpu.sync_copy(dst_hbm.at[pl.ds(base, CHUNK)], dst_v)
            pltpu.sync_copy(rows_v, acc_sh.at[dst_v], add=True)
            return carry

        lax.fori_loop(0, n_full, step, 0)

        if tail:
            src_t, dst_t, rows_t = tail_bufs
            base = e0 + n_full * CHUNK
            pltpu.sync_copy(src_hbm.at[pl.ds(base, tail)], src_t)
            pltpu.async_copy(h_hbm.at[src_t], rows_t, sem).wait()
            pltpu.sync_copy(dst_hbm.at[pl.ds(base, tail)], dst_t)
            pltpu.sync_copy(rows_t, acc_sh.at[dst_t], add=True)

        plsc.subcore_barrier()
        pltpu.sync_copy(acc_sh.at[pl.ds(row0, slab)],
                        out_hbm.at[pl.ds(c * n_nodes + row0, slab)])
        if rem:
            @pl.when(s == NS - 1)
            def _():
                pltpu.sync_copy(
                    acc_sh.at[pl.ds(NS * slab, rem)],
                    out_hbm.at[pl.ds(c * n_nodes + NS * slab, rem)])

    return agg


def kernel(x, adj, W1, b1, W2, b2):
    n, f = x.shape
    h1w = W1.shape[1]
    h2w = W2.shape[1]
    e = adj.shape[1]
    src = adj[0].astype(jnp.int32)
    dst = adj[1].astype(jnp.int32)

    bm = 2000
    h1 = _make_mm_bias(n, f, h1w, bm)(x, W1, b1.reshape(1, h1w))
    z1 = jnp.zeros((n, h1w), jnp.float32)
    p1 = _make_sc_agg(n, e, h1w)(h1, src, dst, z1)
    z2 = _make_fused_relu_mm(n, h1w, h2w, bm)(
        p1[:n], p1[n:], W2, b2.reshape(1, h2w))
    zz = jnp.zeros((n, h2w), jnp.float32)
    p2 = _make_sc_agg(n, e, h2w)(z2, src, dst, zz)
    return _make_add(n, h2w, bm)(p2[:n], p2[n:])


# speedup vs baseline: 9.5368x; 1.6866x over previous
import functools

import jax
import jax.numpy as jnp
from jax import lax
from jax.experimental import pallas as pl
from jax.experimental.pallas import tpu as pltpu
from jax.experimental.pallas import tpu_sc as plsc

NC = 2
NS = 16
NW = NC * NS
CHUNK = 128


def _mm_bias_block(x_ref, w_ref, b_ref, o_ref):
    o_ref[...] = (
        jnp.dot(x_ref[...], w_ref[...], preferred_element_type=jnp.float32)
        + b_ref[...]
    )


@functools.lru_cache(maxsize=None)
def _make_mm_bias(n, k, h, bm):
    return pl.pallas_call(
        _mm_bias_block,
        grid=(n // bm,),
        in_specs=[
            pl.BlockSpec((bm, k), lambda i: (i, 0)),
            pl.BlockSpec((k, h), lambda i: (0, 0)),
            pl.BlockSpec((1, h), lambda i: (0, 0)),
        ],
        out_specs=pl.BlockSpec((bm, h), lambda i: (i, 0)),
        out_shape=jax.ShapeDtypeStruct((n, h), jnp.float32),
    )


def _fused_relu_mm_block(a0_ref, a1_ref, w_ref, b_ref, o_ref):
    h = jnp.maximum(a0_ref[...] + a1_ref[...], 0.0)
    o_ref[...] = (
        jnp.dot(h, w_ref[...], preferred_element_type=jnp.float32) + b_ref[...]
    )


@functools.lru_cache(maxsize=None)
def _make_fused_relu_mm(n, k, h, bm):
    return pl.pallas_call(
        _fused_relu_mm_block,
        grid=(n // bm,),
        in_specs=[
            pl.BlockSpec((bm, k), lambda i: (i, 0)),
            pl.BlockSpec((bm, k), lambda i: (i, 0)),
            pl.BlockSpec((k, h), lambda i: (0, 0)),
            pl.BlockSpec((1, h), lambda i: (0, 0)),
        ],
        out_specs=pl.BlockSpec((bm, h), lambda i: (i, 0)),
        out_shape=jax.ShapeDtypeStruct((n, h), jnp.float32),
    )


def _add_block(a_ref, b_ref, o_ref):
    o_ref[...] = a_ref[...] + b_ref[...]


@functools.lru_cache(maxsize=None)
def _make_add(n, h, bm):
    return pl.pallas_call(
        _add_block,
        grid=(n // bm,),
        in_specs=[
            pl.BlockSpec((bm, h), lambda i: (i, 0)),
            pl.BlockSpec((bm, h), lambda i: (i, 0)),
        ],
        out_specs=pl.BlockSpec((bm, h), lambda i: (i, 0)),
        out_shape=jax.ShapeDtypeStruct((n, h), jnp.float32),
    )


@functools.lru_cache(maxsize=None)
def _make_sc_agg(n_nodes, n_edges, d):
    n_chunks = n_edges // CHUNK
    assert n_chunks * CHUNK == n_edges
    cpw = n_chunks // NW
    n_extra = n_chunks - cpw * NW
    assert cpw % 2 == 1, "pipeline epilogue assumes odd chunks-per-worker"
    npair = (cpw - 1) // 2
    slab = (n_nodes // NS) // 8 * 8
    rem = n_nodes - NS * slab
    assert rem % 8 == 0

    mesh = plsc.VectorSubcoreMesh(core_axis_name="c", subcore_axis_name="s")

    scratch = [
        pltpu.VMEM((cpw + 1, CHUNK), jnp.int32),
        pltpu.VMEM((cpw + 1, CHUNK), jnp.int32),
        pltpu.VMEM((2, CHUNK, d), jnp.float32),
        pltpu.VMEM_SHARED((n_nodes, d), jnp.float32),
        pltpu.SemaphoreType.DMA,
        pltpu.SemaphoreType.DMA,
    ]

    @functools.partial(
        pl.kernel,
        out_type=jax.ShapeDtypeStruct((NC * n_nodes, d), jnp.float32),
        mesh=mesh,
        scratch_types=scratch,
        compiler_params=pltpu.CompilerParams(use_tc_tiling_on_sc=False),
    )
    def agg(h_hbm, src_hbm, dst_hbm, zeros_hbm, out_hbm,
            src_v, dst_v, rows_v, acc_sh, sem0, sem1):
        c = lax.axis_index("c")
        s = lax.axis_index("s")
        wid = s * NC + c
        row0 = s * slab
        c0 = wid * cpw

        pltpu.sync_copy(src_hbm.at[pl.ds(c0, cpw)], src_v.at[pl.ds(0, cpw)])
        pltpu.sync_copy(dst_hbm.at[pl.ds(c0, cpw)], dst_v.at[pl.ds(0, cpw)])
        if n_extra:
            @pl.when(wid < n_extra)
            def _():
                g = NW * cpw + wid
                pltpu.sync_copy(src_hbm.at[pl.ds(g, 1)],
                                src_v.at[pl.ds(cpw, 1)])
                pltpu.sync_copy(dst_hbm.at[pl.ds(g, 1)],
                                dst_v.at[pl.ds(cpw, 1)])

        buf0 = rows_v.at[0]
        buf1 = rows_v.at[1]
        g0 = pltpu.async_copy(h_hbm.at[src_v.at[0]], buf0, sem0)

        pltpu.sync_copy(zeros_hbm.at[pl.ds(row0, slab)],
                        acc_sh.at[pl.ds(row0, slab)])
        if rem:
            @pl.when(s == NS - 1)
            def _():
                pltpu.sync_copy(zeros_hbm.at[pl.ds(NS * slab, rem)],
                                acc_sh.at[pl.ds(NS * slab, rem)])
        plsc.subcore_barrier()

        def step(j, carry):
            cp1 = pltpu.async_copy(h_hbm.at[src_v.at[2 * j + 1]], buf1, sem1)
            g0.wait()
            pltpu.sync_copy(buf0, acc_sh.at[dst_v.at[2 * j]], add=True)
            pltpu.async_copy(h_hbm.at[src_v.at[2 * j + 2]], buf0, sem0)
            cp1.wait()
            pltpu.sync_copy(buf1, acc_sh.at[dst_v.at[2 * j + 1]], add=True)
            return carry

        lax.fori_loop(0, npair, step, 0)
        g0.wait()
        pltpu.sync_copy(buf0, acc_sh.at[dst_v.at[cpw - 1]], add=True)
        if n_extra:
            @pl.when(wid < n_extra)
            def _():
                pltpu.async_copy(h_hbm.at[src_v.at[cpw]], buf1, sem1).wait()
                pltpu.sync_copy(buf1, acc_sh.at[dst_v.at[cpw]], add=True)

        plsc.subcore_barrier()
        pltpu.sync_copy(acc_sh.at[pl.ds(row0, slab)],
                        out_hbm.at[pl.ds(c * n_nodes + row0, slab)])
        if rem:
            @pl.when(s == NS - 1)
            def _():
                pltpu.sync_copy(
                    acc_sh.at[pl.ds(NS * slab, rem)],
                    out_hbm.at[pl.ds(c * n_nodes + NS * slab, rem)])

    return agg


def kernel(x, adj, W1, b1, W2, b2):
    n, f = x.shape
    h1w = W1.shape[1]
    h2w = W2.shape[1]
    e = adj.shape[1]
    src = adj[0].astype(jnp.int32).reshape(e // CHUNK, CHUNK)
    dst = adj[1].astype(jnp.int32).reshape(e // CHUNK, CHUNK)

    bm = 2000
    h1 = _make_mm_bias(n, f, h1w, bm)(x, W1, b1.reshape(1, h1w))
    z1 = jnp.zeros((n, h1w), jnp.float32)
    p1 = _make_sc_agg(n, e, h1w)(h1, src, dst, z1)
    z2 = _make_fused_relu_mm(n, h1w, h2w, bm)(
        p1[:n], p1[n:], W2, b2.reshape(1, h2w))
    zz = jnp.zeros((n, h2w), jnp.float32)
    p2 = _make_sc_agg(n, e, h2w)(z2, src, dst, zz)
    return _make_add(n, h2w, bm)(p2[:n], p2[n:])


# speedup vs baseline: 11.6574x; 1.2224x over previous
import functools

import jax
import jax.numpy as jnp
from jax import lax
from jax.experimental import pallas as pl
from jax.experimental.pallas import tpu as pltpu
from jax.experimental.pallas import tpu_sc as plsc

NC = 2
NS = 16
CHUNK = 128


def _mm_split_block(relu_in, hh, x_ref, w_ref, b_ref, o_ref):
    x = x_ref[...]
    if relu_in:
        x = jnp.maximum(x, 0.0)
    r = jnp.dot(x, w_ref[...], preferred_element_type=jnp.float32) + b_ref[...]
    o_ref[0] = r[:, :hh]
    o_ref[1] = r[:, hh:]


@functools.lru_cache(maxsize=None)
def _make_mm_split(n, k, h, bm, relu_in):
    hh = h // 2
    return pl.pallas_call(
        functools.partial(_mm_split_block, relu_in, hh),
        grid=(n // bm,),
        in_specs=[
            pl.BlockSpec((bm, k), lambda i: (i, 0)),
            pl.BlockSpec((k, h), lambda i: (0, 0)),
            pl.BlockSpec((1, h), lambda i: (0, 0)),
        ],
        out_specs=pl.BlockSpec((2, bm, hh), lambda i: (0, i, 0)),
        out_shape=jax.ShapeDtypeStruct((2, n, hh), jnp.float32),
    )


@functools.lru_cache(maxsize=None)
def _make_sc_agg(n_nodes, n_edges, dh):
    n_chunks = n_edges // CHUNK
    assert n_chunks * CHUNK == n_edges
    cpw = n_chunks // NS
    n_extra = n_chunks - cpw * NS
    NBUF = 4
    assert cpw >= NBUF
    max_cpw = cpw + (1 if n_extra else 0)
    n_groups = -(-max_cpw // NBUF)
    slab = (n_nodes // NS) // 8 * 8
    rem = n_nodes - NS * slab
    assert rem % 8 == 0

    mesh = plsc.VectorSubcoreMesh(core_axis_name="c", subcore_axis_name="s")

    scratch = [
        pltpu.VMEM((cpw + 1, CHUNK), jnp.int32),
        pltpu.VMEM((cpw + 1, CHUNK), jnp.int32),
        pltpu.VMEM((NBUF, CHUNK, dh), jnp.float32),
        pltpu.VMEM_SHARED((n_nodes, dh), jnp.float32),
    ] + [pltpu.SemaphoreType.DMA] * NBUF

    @functools.partial(
        pl.kernel,
        out_type=jax.ShapeDtypeStruct((n_nodes, 2 * dh), jnp.float32),
        mesh=mesh,
        scratch_types=scratch,
        compiler_params=pltpu.CompilerParams(use_tc_tiling_on_sc=False),
    )
    def agg(h_hbm, src_hbm, dst_hbm, zeros_hbm, out_hbm,
            src_v, dst_v, rows_v, acc_sh, *sems):
        c = lax.axis_index("c")
        s = lax.axis_index("s")
        row0 = s * slab
        c0 = s * cpw
        hc = h_hbm.at[c]
        if n_extra:
            cpw_eff = jnp.where(s < n_extra, cpw + 1, cpw)
        else:
            cpw_eff = cpw

        pltpu.sync_copy(src_hbm.at[pl.ds(c0, cpw)], src_v.at[pl.ds(0, cpw)])
        pltpu.sync_copy(dst_hbm.at[pl.ds(c0, cpw)], dst_v.at[pl.ds(0, cpw)])
        if n_extra:
            @pl.when(s < n_extra)
            def _():
                g = NS * cpw + s
                pltpu.sync_copy(src_hbm.at[pl.ds(g, 1)],
                                src_v.at[pl.ds(cpw, 1)])
                pltpu.sync_copy(dst_hbm.at[pl.ds(g, 1)],
                                dst_v.at[pl.ds(cpw, 1)])

        for b in range(NBUF):
            pltpu.async_copy(hc.at[src_v.at[b]], rows_v.at[b], sems[b])

        pltpu.sync_copy(zeros_hbm.at[pl.ds(row0, slab)],
                        acc_sh.at[pl.ds(row0, slab)])
        if rem:
            @pl.when(s == NS - 1)
            def _():
                pltpu.sync_copy(zeros_hbm.at[pl.ds(NS * slab, rem)],
                                acc_sh.at[pl.ds(NS * slab, rem)])
        plsc.subcore_barrier()

        def _wait(b):
            pltpu.make_async_copy(
                hc.at[pl.ds(0, CHUNK)], rows_v.at[b], sems[b]).wait()

        def step(g, carry):
            for b in range(NBUF):
                chunk = g * NBUF + b
                nxt = (g + 1) * NBUF + b

                @pl.when(chunk < cpw_eff)
                def _():
                    _wait(b)
                    pltpu.sync_copy(rows_v.at[b], acc_sh.at[dst_v.at[chunk]],
                                    add=True)

                @pl.when(nxt < cpw_eff)
                def _():
                    pltpu.async_copy(hc.at[src_v.at[nxt]],
                                     rows_v.at[b], sems[b])
            return carry

        lax.fori_loop(0, n_groups, step, 0)

        plsc.subcore_barrier()
        pltpu.sync_copy(acc_sh.at[pl.ds(row0, slab)],
                        out_hbm.at[pl.ds(row0, slab), pl.ds(c * dh, dh)])
        if rem:
            @pl.when(s == NS - 1)
            def _():
                pltpu.sync_copy(
                    acc_sh.at[pl.ds(NS * slab, rem)],
                    out_hbm.at[pl.ds(NS * slab, rem), pl.ds(c * dh, dh)])

    return agg


def kernel(x, adj, W1, b1, W2, b2):
    n, f = x.shape
    h1w = W1.shape[1]
    h2w = W2.shape[1]
    e = adj.shape[1]
    src = adj[0].astype(jnp.int32).reshape(e // CHUNK, CHUNK)
    dst = adj[1].astype(jnp.int32).reshape(e // CHUNK, CHUNK)

    bm = 2000
    h1 = _make_mm_split(n, f, h1w, bm, False)(x, W1, b1.reshape(1, h1w))
    z1 = jnp.zeros((n, h1w // 2), jnp.float32)
    g1 = _make_sc_agg(n, e, h1w // 2)(h1, src, dst, z1)
    z2 = _make_mm_split(n, h1w, h2w, bm, True)(g1, W2, b2.reshape(1, h2w))
    zz = jnp.zeros((n, h2w // 2), jnp.float32)
    return _make_sc_agg(n, e, h2w // 2)(z2, src, dst, zz)


# speedup vs baseline: 11.9207x; 1.0226x over previous
import functools

import jax
import jax.numpy as jnp
from jax import lax
from jax.experimental import pallas as pl
from jax.experimental.pallas import tpu as pltpu
from jax.experimental.pallas import tpu_sc as plsc

NC = 2
NS = 16
NW = NC * NS
CHUNK = 128


def _mm_bias_block(x_ref, w_ref, b_ref, o_ref):
    o_ref[...] = (
        jnp.dot(x_ref[...], w_ref[...], preferred_element_type=jnp.float32)
        + b_ref[...]
    )


@functools.lru_cache(maxsize=None)
def _make_mm_bias(n, k, h, bm):
    return pl.pallas_call(
        _mm_bias_block,
        grid=(n // bm,),
        in_specs=[
            pl.BlockSpec((bm, k), lambda i: (i, 0)),
            pl.BlockSpec((k, h), lambda i: (0, 0)),
            pl.BlockSpec((1, h), lambda i: (0, 0)),
        ],
        out_specs=pl.BlockSpec((bm, h), lambda i: (i, 0)),
        out_shape=jax.ShapeDtypeStruct((n, h), jnp.float32),
    )


def _fused_mm_split_block(hh, p_ref, q_ref, w_ref, b_ref, o_ref):
    a = jnp.maximum(p_ref[...] + q_ref[...], 0.0)
    r = jnp.dot(a, w_ref[...], preferred_element_type=jnp.float32) + b_ref[...]
    o_ref[0] = r[:, :hh]
    o_ref[1] = r[:, hh:]


@functools.lru_cache(maxsize=None)
def _make_fused_mm_split(n, k, h, bm):
    hh = h // 2
    nb = n // bm
    return pl.pallas_call(
        functools.partial(_fused_mm_split_block, hh),
        grid=(nb,),
        in_specs=[
            pl.BlockSpec((bm, k), lambda i: (i, 0)),
            pl.BlockSpec((bm, k), lambda i, _nb=nb: (i + _nb, 0)),
            pl.BlockSpec((k, h), lambda i: (0, 0)),
            pl.BlockSpec((1, h), lambda i: (0, 0)),
        ],
        out_specs=pl.BlockSpec((2, bm, hh), lambda i: (0, i, 0)),
        out_shape=jax.ShapeDtypeStruct((2, n, hh), jnp.float32),
    )


@functools.lru_cache(maxsize=None)
def _make_sc_agg_edge(n_nodes, n_chunks, d, nbuf):
    cpw = -(-n_chunks // NW)
    slab = (n_nodes // NS) // 8 * 8
    rem = n_nodes - NS * slab
    assert rem % 8 == 0

    mesh = plsc.VectorSubcoreMesh(core_axis_name="c", subcore_axis_name="s")
    scratch = [
        pltpu.VMEM((cpw, CHUNK), jnp.int32),
        pltpu.VMEM((cpw, CHUNK), jnp.int32),
        pltpu.VMEM((nbuf, CHUNK, d), jnp.float32),
        pltpu.VMEM_SHARED((n_nodes, d), jnp.float32),
    ] + [pltpu.SemaphoreType.DMA] * nbuf

    @functools.partial(
        pl.kernel,
        out_type=jax.ShapeDtypeStruct((NC * n_nodes, d), jnp.float32),
        mesh=mesh,
        scratch_types=scratch,
    )
    def agg(h_hbm, src_hbm, dst_hbm, zeros_hbm, out_hbm,
            src_v, dst_v, rows_v, acc_sh, *sems):
        c = lax.axis_index("c")
        s = lax.axis_index("s")
        wid = s * NC + c
        row0 = s * slab
        c0 = wid * cpw
        cpw_eff = jnp.clip(n_chunks - c0, 0, cpw)

        pltpu.sync_copy(src_hbm.at[pl.ds(c0, cpw)], src_v)
        pltpu.sync_copy(dst_hbm.at[pl.ds(c0, cpw)], dst_v)

        for b in range(nbuf):
            @pl.when(b < cpw_eff)
            def _():
                pltpu.async_copy(h_hbm.at[src_v.at[b]], rows_v.at[b], sems[b])

        pltpu.sync_copy(zeros_hbm.at[pl.ds(row0, slab)],
                        acc_sh.at[pl.ds(row0, slab)])
        if rem:
            @pl.when(s == NS - 1)
            def _():
                pltpu.sync_copy(zeros_hbm.at[pl.ds(NS * slab, rem)],
                                acc_sh.at[pl.ds(NS * slab, rem)])
        plsc.subcore_barrier()

        def _wait(b):
            pltpu.make_async_copy(
                h_hbm.at[pl.ds(0, CHUNK)], rows_v.at[b], sems[b]).wait()

        n_groups = -(-cpw // nbuf)

        def step(g, carry):
            for b in range(nbuf):
                chunk = g * nbuf + b
                nxt = (g + 1) * nbuf + b

                @pl.when(chunk < cpw_eff)
                def _():
                    _wait(b)
                    pltpu.sync_copy(rows_v.at[b], acc_sh.at[dst_v.at[chunk]],
                                    add=True)

                @pl.when(nxt < cpw_eff)
                def _():
                    pltpu.async_copy(h_hbm.at[src_v.at[nxt]],
                                     rows_v.at[b], sems[b])
            return carry

        lax.fori_loop(0, n_groups, step, 0)

        plsc.subcore_barrier()
        pltpu.sync_copy(acc_sh.at[pl.ds(row0, slab)],
                        out_hbm.at[pl.ds(c * n_nodes + row0, slab)])
        if rem:
            @pl.when(s == NS - 1)
            def _():
                pltpu.sync_copy(
                    acc_sh.at[pl.ds(NS * slab, rem)],
                    out_hbm.at[pl.ds(c * n_nodes + NS * slab, rem)])

    return agg


@functools.lru_cache(maxsize=None)
def _make_sc_agg_feat(n_nodes, n_chunks, dh, nbuf):
    cpw = -(-n_chunks // NS)
    slab = (n_nodes // NS) // 8 * 8
    rem = n_nodes - NS * slab
    assert rem % 8 == 0

    mesh = plsc.VectorSubcoreMesh(core_axis_name="c", subcore_axis_name="s")
    scratch = [
        pltpu.VMEM((cpw, CHUNK), jnp.int32),
        pltpu.VMEM((cpw, CHUNK), jnp.int32),
        pltpu.VMEM((nbuf, CHUNK, dh), jnp.float32),
        pltpu.VMEM_SHARED((n_nodes, dh), jnp.float32),
    ] + [pltpu.SemaphoreType.DMA] * nbuf

    @functools.partial(
        pl.kernel,
        out_type=jax.ShapeDtypeStruct((n_nodes, 2 * dh), jnp.float32),
        mesh=mesh,
        scratch_types=scratch,
        compiler_params=pltpu.CompilerParams(use_tc_tiling_on_sc=False),
    )
    def agg(h_hbm, src_hbm, dst_hbm, zeros_hbm, out_hbm,
            src_v, dst_v, rows_v, acc_sh, *sems):
        c = lax.axis_index("c")
        s = lax.axis_index("s")
        row0 = s * slab
        c0 = s * cpw
        hc = h_hbm.at[c]
        cpw_eff = jnp.clip(n_chunks - c0, 0, cpw)

        pltpu.sync_copy(src_hbm.at[pl.ds(c0, cpw)], src_v)
        pltpu.sync_copy(dst_hbm.at[pl.ds(c0, cpw)], dst_v)

        for b in range(nbuf):
            @pl.when(b < cpw_eff)
            def _():
                pltpu.async_copy(hc.at[src_v.at[b]], rows_v.at[b], sems[b])

        pltpu.sync_copy(zeros_hbm.at[pl.ds(row0, slab)],
                        acc_sh.at[pl.ds(row0, slab)])
        if rem:
            @pl.when(s == NS - 1)
            def _():
                pltpu.sync_copy(zeros_hbm.at[pl.ds(NS * slab, rem)],
                                acc_sh.at[pl.ds(NS * slab, rem)])
        plsc.subcore_barrier()

        def _wait(b):
            pltpu.make_async_copy(
                hc.at[pl.ds(0, CHUNK)], rows_v.at[b], sems[b]).wait()

        n_groups = -(-cpw // nbuf)

        def step(g, carry):
            for b in range(nbuf):
                chunk = g * nbuf + b
                nxt = (g + 1) * nbuf + b

                @pl.when(chunk < cpw_eff)
                def _():
                    _wait(b)
                    pltpu.sync_copy(rows_v.at[b], acc_sh.at[dst_v.at[chunk]],
                                    add=True)

                @pl.when(nxt < cpw_eff)
                def _():
                    pltpu.async_copy(hc.at[src_v.at[nxt]],
                                     rows_v.at[b], sems[b])
            return carry

        lax.fori_loop(0, n_groups, step, 0)

        plsc.subcore_barrier()
        pltpu.sync_copy(acc_sh.at[pl.ds(row0, slab)],
                        out_hbm.at[pl.ds(row0, slab), pl.ds(c * dh, dh)])
        if rem:
            @pl.when(s == NS - 1)
            def _():
                pltpu.sync_copy(
                    acc_sh.at[pl.ds(NS * slab, rem)],
                    out_hbm.at[pl.ds(NS * slab, rem), pl.ds(c * dh, dh)])

    return agg


def kernel(x, adj, W1, b1, W2, b2):
    n, f = x.shape
    h1w = W1.shape[1]
    h2w = W2.shape[1]
    e = adj.shape[1]
    n_chunks = e // CHUNK
    pad_chunks = -(-n_chunks // NW) * NW
    idx = adj.astype(jnp.int32).reshape(2, n_chunks, CHUNK)
    idx = jnp.pad(idx, ((0, 0), (0, pad_chunks - n_chunks), (0, 0)))
    src, dst = idx[0], idx[1]

    bm = 2000
    h1 = _make_mm_bias(n, f, h1w, bm)(x, W1, b1.reshape(1, h1w))
    z1 = jnp.zeros((n, h1w), jnp.float32)
    p = _make_sc_agg_edge(n, n_chunks, h1w, 2)(h1, src, dst, z1)
    z2 = _make_fused_mm_split(n, h1w, h2w, bm)(p, p, W2, b2.reshape(1, h2w))
    zz = jnp.zeros((n, h2w // 2), jnp.float32)
    return _make_sc_agg_feat(n, n_chunks, h2w // 2, 4)(z2, src, dst, zz)


# speedup vs baseline: 12.1153x; 1.0163x over previous
import functools

import jax
import jax.numpy as jnp
from jax import lax
from jax.experimental import pallas as pl
from jax.experimental.pallas import tpu as pltpu
from jax.experimental.pallas import tpu_sc as plsc

NC = 2
NS = 16
NW = NC * NS
CHUNK = 128


def _mm_bias_block(x_ref, w_ref, b_ref, o_ref):
    o_ref[...] = (
        jnp.dot(x_ref[...], w_ref[...], preferred_element_type=jnp.float32)
        + b_ref[...]
    )


@functools.lru_cache(maxsize=None)
def _make_mm_bias(n, k, h, bm):
    return pl.pallas_call(
        _mm_bias_block,
        grid=(n // bm,),
        in_specs=[
            pl.BlockSpec((bm, k), lambda i: (i, 0)),
            pl.BlockSpec((k, h), lambda i: (0, 0)),
            pl.BlockSpec((1, h), lambda i: (0, 0)),
        ],
        out_specs=pl.BlockSpec((bm, h), lambda i: (i, 0)),
        out_shape=jax.ShapeDtypeStruct((n, h), jnp.float32),
    )


def _fused_mm_split_block(hh, p_ref, q_ref, w_ref, b_ref, o_ref):
    a = jnp.maximum(p_ref[...] + q_ref[...], 0.0)
    r = jnp.dot(a, w_ref[...], preferred_element_type=jnp.float32) + b_ref[...]
    o_ref[0] = r[:, :hh]
    o_ref[1] = r[:, hh:]


@functools.lru_cache(maxsize=None)
def _make_fused_mm_split(n, k, h, bm):
    hh = h // 2
    nb = n // bm
    return pl.pallas_call(
        functools.partial(_fused_mm_split_block, hh),
        grid=(nb,),
        in_specs=[
            pl.BlockSpec((bm, k), lambda i: (i, 0)),
            pl.BlockSpec((bm, k), lambda i, _nb=nb: (i + _nb, 0)),
            pl.BlockSpec((k, h), lambda i: (0, 0)),
            pl.BlockSpec((1, h), lambda i: (0, 0)),
        ],
        out_specs=pl.BlockSpec((2, bm, hh), lambda i: (0, i, 0)),
        out_shape=jax.ShapeDtypeStruct((2, n, hh), jnp.float32),
    )


@functools.lru_cache(maxsize=None)
def _make_sc_agg_edge(n_nodes, n_chunks, d, nbuf):
    cpw = -(-n_chunks // NW)
    slab = (n_nodes // NS) // 8 * 8
    rem = n_nodes - NS * slab
    assert rem % 8 == 0

    mesh = plsc.VectorSubcoreMesh(core_axis_name="c", subcore_axis_name="s")
    scratch = [
        pltpu.VMEM((cpw, CHUNK), jnp.int32),
        pltpu.VMEM((cpw, CHUNK), jnp.int32),
        pltpu.VMEM((nbuf, CHUNK, d), jnp.float32),
        pltpu.VMEM_SHARED((n_nodes, d), jnp.float32),
    ] + [pltpu.SemaphoreType.DMA] * nbuf

    @functools.partial(
        pl.kernel,
        out_type=jax.ShapeDtypeStruct((NC * n_nodes, d), jnp.float32),
        mesh=mesh,
        scratch_types=scratch,
    )
    def agg(h_hbm, src_hbm, dst_hbm, zeros_hbm, out_hbm,
            src_v, dst_v, rows_v, acc_sh, *sems):
        c = lax.axis_index("c")
        s = lax.axis_index("s")
        wid = s * NC + c
        row0 = s * slab
        c0 = wid * cpw
        cpw_eff = jnp.clip(n_chunks - c0, 0, cpw)

        pltpu.sync_copy(src_hbm.at[pl.ds(c0, cpw)], src_v)
        pltpu.sync_copy(dst_hbm.at[pl.ds(c0, cpw)], dst_v)

        for b in range(nbuf):
            @pl.when(b < cpw_eff)
            def _():
                pltpu.async_copy(h_hbm.at[src_v.at[b]], rows_v.at[b], sems[b])

        pltpu.sync_copy(zeros_hbm.at[pl.ds(row0, slab)],
                        acc_sh.at[pl.ds(row0, slab)])
        if rem:
            @pl.when(s == NS - 1)
            def _():
                pltpu.sync_copy(zeros_hbm.at[pl.ds(NS * slab, rem)],
                                acc_sh.at[pl.ds(NS * slab, rem)])
        plsc.subcore_barrier()

        def _wait(b):
            pltpu.make_async_copy(
                h_hbm.at[pl.ds(0, CHUNK)], rows_v.at[b], sems[b]).wait()

        n_groups = -(-cpw // nbuf)

        def step(g, carry):
            for b in range(nbuf):
                chunk = g * nbuf + b
                nxt = (g + 1) * nbuf + b

                @pl.when(chunk < cpw_eff)
                def _():
                    _wait(b)
                    pltpu.sync_copy(rows_v.at[b], acc_sh.at[dst_v.at[chunk]],
                                    add=True)

                @pl.when(nxt < cpw_eff)
                def _():
                    pltpu.async_copy(h_hbm.at[src_v.at[nxt]],
                                     rows_v.at[b], sems[b])
            return carry

        lax.fori_loop(0, n_groups, step, 0)

        plsc.subcore_barrier()
        pltpu.sync_copy(acc_sh.at[pl.ds(row0, slab)],
                        out_hbm.at[pl.ds(c * n_nodes + row0, slab)])
        if rem:
            @pl.when(s == NS - 1)
            def _():
                pltpu.sync_copy(
                    acc_sh.at[pl.ds(NS * slab, rem)],
                    out_hbm.at[pl.ds(c * n_nodes + NS * slab, rem)])

    return agg


@functools.lru_cache(maxsize=None)
def _make_sc_agg_feat(n_nodes, n_chunks, dh, nbuf):
    cpw = -(-n_chunks // NS)
    slab = (n_nodes // NS) // 8 * 8
    rem = n_nodes - NS * slab
    assert rem % 8 == 0

    mesh = plsc.VectorSubcoreMesh(core_axis_name="c", subcore_axis_name="s")
    scratch = [
        pltpu.VMEM((cpw, CHUNK), jnp.int32),
        pltpu.VMEM((cpw, CHUNK), jnp.int32),
        pltpu.VMEM((nbuf, CHUNK, dh), jnp.float32),
        pltpu.VMEM_SHARED((n_nodes, dh), jnp.float32),
    ] + [pltpu.SemaphoreType.DMA] * nbuf

    @functools.partial(
        pl.kernel,
        out_type=jax.ShapeDtypeStruct((n_nodes, 2 * dh), jnp.float32),
        mesh=mesh,
        scratch_types=scratch,
        compiler_params=pltpu.CompilerParams(use_tc_tiling_on_sc=False),
    )
    def agg(h_hbm, src_hbm, dst_hbm, zeros_hbm, out_hbm,
            src_v, dst_v, rows_v, acc_sh, *sems):
        c = lax.axis_index("c")
        s = lax.axis_index("s")
        row0 = s * slab
        c0 = s * cpw
        hc = h_hbm.at[c]
        cpw_eff = jnp.clip(n_chunks - c0, 0, cpw)

        pltpu.sync_copy(src_hbm.at[pl.ds(c0, cpw)], src_v)
        pltpu.sync_copy(dst_hbm.at[pl.ds(c0, cpw)], dst_v)

        for b in range(nbuf):
            @pl.when(b < cpw_eff)
            def _():
                pltpu.async_copy(hc.at[src_v.at[b]], rows_v.at[b], sems[b])

        pltpu.sync_copy(zeros_hbm.at[pl.ds(row0, slab)],
                        acc_sh.at[pl.ds(row0, slab)])
        if rem:
            @pl.when(s == NS - 1)
            def _():
                pltpu.sync_copy(zeros_hbm.at[pl.ds(NS * slab, rem)],
                                acc_sh.at[pl.ds(NS * slab, rem)])
        plsc.subcore_barrier()

        def _wait(b):
            pltpu.make_async_copy(
                hc.at[pl.ds(0, CHUNK)], rows_v.at[b], sems[b]).wait()

        n_groups = -(-cpw // nbuf)

        def step(g, carry):
            for b in range(nbuf):
                chunk = g * nbuf + b
                nxt = (g + 1) * nbuf + b

                @pl.when(chunk < cpw_eff)
                def _():
                    _wait(b)
                    pltpu.sync_copy(rows_v.at[b], acc_sh.at[dst_v.at[chunk]],
                                    add=True)

                @pl.when(nxt < cpw_eff)
                def _():
                    pltpu.async_copy(hc.at[src_v.at[nxt]],
                                     rows_v.at[b], sems[b])
            return carry

        lax.fori_loop(0, n_groups, step, 0)

        plsc.subcore_barrier()
        pltpu.sync_copy(acc_sh.at[pl.ds(row0, slab)],
                        out_hbm.at[pl.ds(row0, slab), pl.ds(c * dh, dh)])
        if rem:
            @pl.when(s == NS - 1)
            def _():
                pltpu.sync_copy(
                    acc_sh.at[pl.ds(NS * slab, rem)],
                    out_hbm.at[pl.ds(NS * slab, rem), pl.ds(c * dh, dh)])

    return agg


def kernel(x, adj, W1, b1, W2, b2):
    n, f = x.shape
    h1w = W1.shape[1]
    h2w = W2.shape[1]
    e = adj.shape[1]
    n_chunks = e // CHUNK
    pad_chunks = -(-n_chunks // NW) * NW
    idx = adj.astype(jnp.int32).reshape(2, n_chunks, CHUNK)
    idx = jnp.pad(idx, ((0, 0), (0, pad_chunks - n_chunks), (0, 0)))
    src, dst = idx[0], idx[1]

    bm = 2000
    h1 = _make_mm_bias(n, f, h1w, bm)(x, W1, b1.reshape(1, h1w))
    z1 = jnp.zeros((n, h1w), jnp.float32)
    p = _make_sc_agg_edge(n, n_chunks, h1w, 2)(h1, src, dst, z1)
    z2 = _make_fused_mm_split(n, h1w, h2w, bm)(p, p, W2, b2.reshape(1, h2w))
    zz = jnp.zeros((n, h2w // 2), jnp.float32)
    return _make_sc_agg_feat(n, n_chunks, h2w // 2, 6)(z2, src, dst, zz)


# speedup vs baseline: 12.4180x; 1.0250x over previous
import functools

import jax
import jax.numpy as jnp
from jax import lax
from jax.experimental import pallas as pl
from jax.experimental.pallas import tpu as pltpu
from jax.experimental.pallas import tpu_sc as plsc

NC = 2
NS = 16
NW = NC * NS
CHUNK = 128


def _mm_bias_block(x_ref, w_ref, b_ref, o_ref):
    o_ref[...] = (
        jnp.dot(x_ref[...], w_ref[...], preferred_element_type=jnp.float32)
        + b_ref[...]
    )


@functools.lru_cache(maxsize=None)
def _make_mm_bias(n, k, h, bm):
    return pl.pallas_call(
        _mm_bias_block,
        grid=(n // bm,),
        in_specs=[
            pl.BlockSpec((bm, k), lambda i: (i, 0)),
            pl.BlockSpec((k, h), lambda i: (0, 0)),
            pl.BlockSpec((1, h), lambda i: (0, 0)),
        ],
        out_specs=pl.BlockSpec((bm, h), lambda i: (i, 0)),
        out_shape=jax.ShapeDtypeStruct((n, h), jnp.float32),
    )


def _mm_split_block(relu_in, hh, x_ref, w_ref, b_ref, o_ref):
    x = x_ref[...]
    if relu_in:
        x = jnp.maximum(x, 0.0)
    r = jnp.dot(x, w_ref[...], preferred_element_type=jnp.float32) + b_ref[...]
    o_ref[0] = r[:, :hh]
    o_ref[1] = r[:, hh:]


@functools.lru_cache(maxsize=None)
def _make_mm_split(n, k, h, bm, relu_in):
    hh = h // 2
    return pl.pallas_call(
        functools.partial(_mm_split_block, relu_in, hh),
        grid=(n // bm,),
        in_specs=[
            pl.BlockSpec((bm, k), lambda i: (i, 0)),
            pl.BlockSpec((k, h), lambda i: (0, 0)),
            pl.BlockSpec((1, h), lambda i: (0, 0)),
        ],
        out_specs=pl.BlockSpec((2, bm, hh), lambda i: (0, i, 0)),
        out_shape=jax.ShapeDtypeStruct((2, n, hh), jnp.float32),
    )


def _fused_mm_split_block(hh, p_ref, q_ref, w_ref, b_ref, o_ref):
    a = jnp.maximum(p_ref[...] + q_ref[...], 0.0)
    r = jnp.dot(a, w_ref[...], preferred_element_type=jnp.float32) + b_ref[...]
    o_ref[0] = r[:, :hh]
    o_ref[1] = r[:, hh:]


@functools.lru_cache(maxsize=None)
def _make_fused_mm_split(n, k, h, bm):
    hh = h // 2
    nb = n // bm
    return pl.pallas_call(
        functools.partial(_fused_mm_split_block, hh),
        grid=(nb,),
        in_specs=[
            pl.BlockSpec((bm, k), lambda i: (i, 0)),
            pl.BlockSpec((bm, k), lambda i, _nb=nb: (i + _nb, 0)),
            pl.BlockSpec((k, h), lambda i: (0, 0)),
            pl.BlockSpec((1, h), lambda i: (0, 0)),
        ],
        out_specs=pl.BlockSpec((2, bm, hh), lambda i: (0, i, 0)),
        out_shape=jax.ShapeDtypeStruct((2, n, hh), jnp.float32),
    )


@functools.lru_cache(maxsize=None)
def _make_sc_agg_edge(n_nodes, n_chunks, d, nbuf):
    cpw = -(-n_chunks // NW)
    slab = (n_nodes // NS) // 8 * 8
    rem = n_nodes - NS * slab
    assert rem % 8 == 0

    mesh = plsc.VectorSubcoreMesh(core_axis_name="c", subcore_axis_name="s")
    scratch = [
        pltpu.VMEM((cpw, CHUNK), jnp.int32),
        pltpu.VMEM((cpw, CHUNK), jnp.int32),
        pltpu.VMEM((nbuf, CHUNK, d), jnp.float32),
        pltpu.VMEM_SHARED((n_nodes, d), jnp.float32),
    ] + [pltpu.SemaphoreType.DMA] * nbuf

    @functools.partial(
        pl.kernel,
        out_type=jax.ShapeDtypeStruct((NC * n_nodes, d), jnp.float32),
        mesh=mesh,
        scratch_types=scratch,
    )
    def agg(h_hbm, src_hbm, dst_hbm, zeros_hbm, out_hbm,
            src_v, dst_v, rows_v, acc_sh, *sems):
        c = lax.axis_index("c")
        s = lax.axis_index("s")
        wid = s * NC + c
        row0 = s * slab
        c0 = wid * cpw
        cpw_eff = jnp.clip(n_chunks - c0, 0, cpw)

        pltpu.sync_copy(src_hbm.at[pl.ds(c0, cpw)], src_v)
        pltpu.sync_copy(dst_hbm.at[pl.ds(c0, cpw)], dst_v)

        for b in range(nbuf):
            @pl.when(b < cpw_eff)
            def _():
                pltpu.async_copy(h_hbm.at[src_v.at[b]], rows_v.at[b], sems[b])

        pltpu.sync_copy(zeros_hbm.at[pl.ds(row0, slab)],
                        acc_sh.at[pl.ds(row0, slab)])
        if rem:
            @pl.when(s == NS - 1)
            def _():
                pltpu.sync_copy(zeros_hbm.at[pl.ds(NS * slab, rem)],
                                acc_sh.at[pl.ds(NS * slab, rem)])
        plsc.subcore_barrier()

        def _wait(b):
            pltpu.make_async_copy(
                h_hbm.at[pl.ds(0, CHUNK)], rows_v.at[b], sems[b]).wait()

        n_groups = -(-cpw // nbuf)

        def step(g, carry):
            for b in range(nbuf):
                chunk = g * nbuf + b
                nxt = (g + 1) * nbuf + b

                @pl.when(chunk < cpw_eff)
                def _():
                    _wait(b)
                    pltpu.sync_copy(rows_v.at[b], acc_sh.at[dst_v.at[chunk]],
                                    add=True)

                @pl.when(nxt < cpw_eff)
                def _():
                    pltpu.async_copy(h_hbm.at[src_v.at[nxt]],
                                     rows_v.at[b], sems[b])
            return carry

        lax.fori_loop(0, n_groups, step, 0)

        plsc.subcore_barrier()
        pltpu.sync_copy(acc_sh.at[pl.ds(row0, slab)],
                        out_hbm.at[pl.ds(c * n_nodes + row0, slab)])
        if rem:
            @pl.when(s == NS - 1)
            def _():
                pltpu.sync_copy(
                    acc_sh.at[pl.ds(NS * slab, rem)],
                    out_hbm.at[pl.ds(c * n_nodes + NS * slab, rem)])

    return agg


@functools.lru_cache(maxsize=None)
def _make_sc_agg_feat(n_nodes, n_chunks, dh, nbuf):
    cpw = -(-n_chunks // NS)
    slab = (n_nodes // NS) // 8 * 8
    rem = n_nodes - NS * slab
    assert rem % 8 == 0

    mesh = plsc.VectorSubcoreMesh(core_axis_name="c", subcore_axis_name="s")
    scratch = [
        pltpu.VMEM((cpw, CHUNK), jnp.int32),
        pltpu.VMEM((cpw, CHUNK), jnp.int32),
        pltpu.VMEM((nbuf, CHUNK, dh), jnp.float32),
        pltpu.VMEM_SHARED((n_nodes, dh), jnp.float32),
    ] + [pltpu.SemaphoreType.DMA] * nbuf

    @functools.partial(
        pl.kernel,
        out_type=jax.ShapeDtypeStruct((n_nodes, 2 * dh), jnp.float32),
        mesh=mesh,
        scratch_types=scratch,
        compiler_params=pltpu.CompilerParams(use_tc_tiling_on_sc=False),
    )
    def agg(h_hbm, src_hbm, dst_hbm, zeros_hbm, out_hbm,
            src_v, dst_v, rows_v, acc_sh, *sems):
        c = lax.axis_index("c")
        s = lax.axis_index("s")
        row0 = s * slab
        c0 = s * cpw
        hc = h_hbm.at[c]
        cpw_eff = jnp.clip(n_chunks - c0, 0, cpw)

        pltpu.sync_copy(src_hbm.at[pl.ds(c0, cpw)], src_v)
        pltpu.sync_copy(dst_hbm.at[pl.ds(c0, cpw)], dst_v)

        for b in range(nbuf):
            @pl.when(b < cpw_eff)
            def _():
                pltpu.async_copy(hc.at[src_v.at[b]], rows_v.at[b], sems[b])

        pltpu.sync_copy(zeros_hbm.at[pl.ds(row0, slab)],
                        acc_sh.at[pl.ds(row0, slab)])
        if rem:
            @pl.when(s == NS - 1)
            def _():
                pltpu.sync_copy(zeros_hbm.at[pl.ds(NS * slab, rem)],
                                acc_sh.at[pl.ds(NS * slab, rem)])
        plsc.subcore_barrier()

        def _wait(b):
            pltpu.make_async_copy(
                hc.at[pl.ds(0, CHUNK)], rows_v.at[b], sems[b]).wait()

        n_groups = -(-cpw // nbuf)

        def step(g, carry):
            for b in range(nbuf):
                chunk = g * nbuf + b
                nxt = (g + 1) * nbuf + b

                @pl.when(chunk < cpw_eff)
                def _():
                    _wait(b)
                    pltpu.sync_copy(rows_v.at[b], acc_sh.at[dst_v.at[chunk]],
                                    add=True)

                @pl.when(nxt < cpw_eff)
                def _():
                    pltpu.async_copy(hc.at[src_v.at[nxt]],
                                     rows_v.at[b], sems[b])
            return carry

        lax.fori_loop(0, n_groups, step, 0)

        plsc.subcore_barrier()
        pltpu.sync_copy(acc_sh.at[pl.ds(row0, slab)],
                        out_hbm.at[pl.ds(row0, slab), pl.ds(c * dh, dh)])
        if rem:
            @pl.when(s == NS - 1)
            def _():
                pltpu.sync_copy(
                    acc_sh.at[pl.ds(NS * slab, rem)],
                    out_hbm.at[pl.ds(NS * slab, rem), pl.ds(c * dh, dh)])

    return agg


def kernel(x, adj, W1, b1, W2, b2):
    n, f = x.shape
    h1w = W1.shape[1]
    h2w = W2.shape[1]
    e = adj.shape[1]
    n_chunks = e // CHUNK
    pad_chunks = -(-n_chunks // NW) * NW
    idx = adj.astype(jnp.int32).reshape(2, n_chunks, CHUNK)
    idx = jnp.pad(idx, ((0, 0), (0, pad_chunks - n_chunks), (0, 0)))
    src, dst = idx[0], idx[1]

    bm = 2000
    h1 = _make_mm_split(n, f, h1w, bm, False)(x, W1, b1.reshape(1, h1w))
    z1 = jnp.zeros((n, h1w // 2), jnp.float32)
    g1 = _make_sc_agg_feat(n, n_chunks, h1w // 2, 6)(h1, src, dst, z1)
    z2 = _make_mm_split(n, h1w, h2w, bm, True)(g1, W2, b2.reshape(1, h2w))
    zz = jnp.zeros((n, h2w // 2), jnp.float32)
    return _make_sc_agg_feat(n, n_chunks, h2w // 2, 6)(z2, src, dst, zz)
